# baseline (speedup 1.0000x reference)
import dataclasses
import functools

import jax
import jax.numpy as jnp
from jax import lax
from jax.experimental import pallas as pl
from jax.experimental.pallas import tpu as pltpu
from jax.experimental.pallas import tpu_sc as plsc


_TILE = 4096
_B = 16384
_NC, _NS, _L = 2, 16, 16
_CHUNK = _B // (_NC * _NS)


def _scores_kernel(obs_ref, w0_ref, b0_ref, w1_ref, b1_ref,
                   w2_ref, b2_ref, s_ref):
    h0 = jnp.maximum(
        jnp.dot(obs_ref[...], w0_ref[...], preferred_element_type=jnp.float32)
        + b0_ref[...], 0.0).astype(jnp.bfloat16)
    h1 = jnp.maximum(
        jnp.dot(h0, w1_ref[...], preferred_element_type=jnp.float32)
        .astype(jnp.bfloat16) + b1_ref[...], jnp.bfloat16(0))
    s_ref[...] = (jnp.dot(h1, w2_ref[...], preferred_element_type=jnp.float32)
                  + b2_ref[...])


def _run_scores(obs_vec, W0, b0, W1cat, b1cat, W2blk, b2cat):
    B = obs_vec.shape[0]
    tile = _TILE
    grid = (B // tile,)
    return pl.pallas_call(
        _scores_kernel,
        grid=grid,
        in_specs=[
            pl.BlockSpec((tile, 10), lambda i: (i, 0)),
            pl.BlockSpec((10, 32), lambda i: (0, 0)),
            pl.BlockSpec((1, 32), lambda i: (0, 0)),
            pl.BlockSpec((32, 512), lambda i: (0, 0)),
            pl.BlockSpec((1, 512), lambda i: (0, 0)),
            pl.BlockSpec((512, 32), lambda i: (0, 0)),
            pl.BlockSpec((1, 32), lambda i: (0, 0)),
        ],
        out_specs=pl.BlockSpec((tile, 32), lambda i: (i, 0)),
        out_shape=jax.ShapeDtypeStruct((B, 32), jnp.float32),
    )(obs_vec.astype(jnp.bfloat16), W0.astype(jnp.bfloat16),
      b0.reshape(1, 32).astype(jnp.bfloat16), W1cat.astype(jnp.bfloat16),
      b1cat.astype(jnp.bfloat16), W2blk.astype(jnp.bfloat16), b2cat)


def _sc_select(s, z):
    mesh = plsc.VectorSubcoreMesh(core_axis_name="c", subcore_axis_name="s")
    cp = pltpu.CompilerParams()
    if "needs_layout_passes" in pltpu.CompilerParams.__dataclass_fields__:
        cp = dataclasses.replace(cp, needs_layout_passes=False)

    @functools.partial(
        pl.kernel, mesh=mesh, compiler_params=cp,
        out_type=jax.ShapeDtypeStruct((2, _B), jnp.float32),
        scratch_types=[
            pltpu.VMEM((_CHUNK, 32), jnp.float32),
            pltpu.VMEM((_CHUNK,), jnp.int32),
            pltpu.VMEM((_CHUNK,), jnp.float32),
            pltpu.VMEM((_CHUNK,), jnp.float32),
        ],
    )
    def sel(s_hbm, z_hbm, out_hbm, s_v, z_v, x_v, y_v):
        wid = lax.axis_index("s") * _NC + lax.axis_index("c")
        base = wid * _CHUNK
        pltpu.sync_copy(s_hbm.at[pl.ds(base, _CHUNK)], s_v)
        pltpu.sync_copy(z_hbm.at[pl.ds(base, _CHUNK)], z_v)

        @pl.loop(0, _CHUNK, step=_L)
        def _(t):
            rows = lax.iota(jnp.int32, _L) + t
            zv = z_v[pl.ds(t, _L)]
            x_v[pl.ds(t, _L)] = plsc.load_gather(s_v, [rows, zv])
            y_v[pl.ds(t, _L)] = plsc.load_gather(s_v, [rows, zv + 16])

        pltpu.sync_copy(x_v, out_hbm.at[0, pl.ds(base, _CHUNK)])
        pltpu.sync_copy(y_v, out_hbm.at[1, pl.ds(base, _CHUNK)])

    return sel(s, z)


def kernel(obs_vec, z_logits, best, W0, b0, Wx1, bx1, Wx2, bx2, Wy1, by1, Wy2, by2):
    n_modes = Wx1.shape[0]
    hid = Wx1.shape[2]
    W1x = Wx1.transpose(1, 0, 2).reshape(32, n_modes * hid)
    W1y = Wy1.transpose(1, 0, 2).reshape(32, n_modes * hid)
    W1cat = jnp.concatenate([W1x, W1y], axis=1)
    b1cat = jnp.concatenate([bx1.reshape(1, -1), by1.reshape(1, -1)], axis=1)
    ex = jnp.eye(n_modes, dtype=jnp.float32)
    w2x = Wx2[:, :, 0]
    w2y = Wy2[:, :, 0]
    blk_x = (ex[:, None, :] * w2x[:, :, None]).reshape(n_modes * hid, n_modes)
    blk_y = (ex[:, None, :] * w2y[:, :, None]).reshape(n_modes * hid, n_modes)
    zeros = jnp.zeros_like(blk_x)
    W2blk = jnp.concatenate(
        [jnp.concatenate([blk_x, zeros], axis=1),
         jnp.concatenate([zeros, blk_y], axis=1)], axis=0)
    b2cat = jnp.concatenate([bx2[:, 0], by2[:, 0]]).reshape(1, 32)

    s = _run_scores(obs_vec, W0, b0, W1cat, b1cat, W2blk, b2cat)
    out2 = _sc_select(s, z_logits.astype(jnp.int32))
    actions = out2.T
    return (actions, z_logits)


# speedup vs baseline: 1.3903x; 1.3903x over previous
import jax
import jax.numpy as jnp
from jax.experimental import pallas as pl


_TILE = 4096


def _fused_kernel(obs_ref, z_ref, w0_ref, b0_ref, w1_ref, b1_ref,
                  w2_ref, b2_ref, out_ref):
    h0 = jnp.maximum(
        jnp.dot(obs_ref[...], w0_ref[...], preferred_element_type=jnp.float32)
        + b0_ref[...], 0.0).astype(jnp.bfloat16)
    h1 = jnp.maximum(
        jnp.dot(h0, w1_ref[...], preferred_element_type=jnp.float32)
        .astype(jnp.bfloat16) + b1_ref[...], jnp.bfloat16(0))
    s = (jnp.dot(h1, w2_ref[...], preferred_element_type=jnp.float32)
         + b2_ref[...])
    m = z_ref[...]
    lane = jax.lax.broadcasted_iota(jnp.int32, (1, 32), 1)
    mask = ((lane & 15) == m).astype(jnp.float32)
    e = jnp.concatenate(
        [jnp.where(lane < 16, 1.0, 0.0).reshape(32, 1),
         jnp.where(lane >= 16, 1.0, 0.0).reshape(32, 1)], axis=1)
    out_ref[...] = jnp.dot(s * mask, e, preferred_element_type=jnp.float32)


def _run(obs_vec, z2d, W0, b0, W1cat, b1cat, W2blk, b2cat):
    B = obs_vec.shape[0]
    tile = _TILE
    grid = (B // tile,)
    return pl.pallas_call(
        _fused_kernel,
        grid=grid,
        in_specs=[
            pl.BlockSpec((tile, 10), lambda i: (i, 0)),
            pl.BlockSpec((tile, 1), lambda i: (i, 0)),
            pl.BlockSpec((10, 32), lambda i: (0, 0)),
            pl.BlockSpec((1, 32), lambda i: (0, 0)),
            pl.BlockSpec((32, 512), lambda i: (0, 0)),
            pl.BlockSpec((1, 512), lambda i: (0, 0)),
            pl.BlockSpec((512, 32), lambda i: (0, 0)),
            pl.BlockSpec((1, 32), lambda i: (0, 0)),
        ],
        out_specs=pl.BlockSpec((tile, 2), lambda i: (i, 0)),
        out_shape=jax.ShapeDtypeStruct((B, 2), jnp.float32),
    )(obs_vec, z2d, W0, b0, W1cat, b1cat, W2blk, b2cat)


_BLKMASK = jnp.repeat(jnp.eye(32, dtype=jnp.float32), 16, axis=0)


def kernel(obs_vec, z_logits, best, W0, b0, Wx1, bx1, Wx2, bx2, Wy1, by1, Wy2, by2):
    n_modes = Wx1.shape[0]
    hid = Wx1.shape[2]
    W1cat = (jnp.concatenate([Wx1, Wy1], axis=0)
             .transpose(1, 0, 2).reshape(32, 2 * n_modes * hid))
    b1cat = jnp.concatenate([bx1, by1], axis=0).reshape(1, -1)
    w2flat = jnp.concatenate([Wx2[:, :, 0], Wy2[:, :, 0]], axis=0).reshape(-1)
    W2blk = _BLKMASK * w2flat[:, None]
    b2cat = jnp.concatenate([bx2[:, 0], by2[:, 0]]).reshape(1, 32)

    actions = _run(
        obs_vec.astype(jnp.bfloat16), z_logits.reshape(-1, 1),
        W0.astype(jnp.bfloat16), b0.reshape(1, 32).astype(jnp.bfloat16),
        W1cat.astype(jnp.bfloat16), b1cat.astype(jnp.bfloat16),
        W2blk.astype(jnp.bfloat16), b2cat)
    return (actions, z_logits)
